# baseline (speedup 1.0000x reference)
import jax
import jax.numpy as jnp
from jax import lax
from jax.experimental import pallas as pl
from jax.experimental.pallas import tpu as pltpu
from jax.experimental.pallas import tpu_sc as plsc

_K = 10
_TEMPERATURE = 1.0
_ROWS = 16384
_N = 1000
_LANES = 16
_NCHUNK = 63
_NWORKERS = 32
_RPW = _ROWS // _NWORKERS
_BATCH = 16
_NBATCH = _RPW // _BATCH
_REGION = 64
_STATIC = 8


def _allreduce(v, op, lanes):
  for sh in (8, 4, 2, 1):
    v = op(v, v[jnp.bitwise_xor(lanes, sh)])
  return v


def _body(p_hbm, r_hbm, w_hbm, out_hbm,
          pbuf0, pbuf1, rbuf0, rbuf1, cand_i, wbuf, obuf, iobuf, sems):
  pbufs = (pbuf0, pbuf1)
  rbufs = (rbuf0, rbuf1)
  wid = lax.axis_index("s") * 2 + lax.axis_index("c")
  base = wid * _RPW

  lanes = lax.iota(jnp.int32, _LANES)
  posb = lanes * _REGION
  tail_mask = lanes < 8
  minf = jnp.full((_LANES,), -jnp.inf, jnp.float32)
  neg1 = jnp.full((_LANES,), -1.0, jnp.float32)
  izero = jnp.zeros((_LANES,), jnp.int32)

  cols = [c * _LANES + lanes for c in range(_NCHUNK - 1)]
  cols.append(jnp.minimum((_NCHUNK - 1) * _LANES + lanes, _N - 1))

  pltpu.sync_copy(w_hbm, wbuf)
  wvec = wbuf[...]

  def copy_batch(j, par):
    r0 = base + j * _BATCH
    pltpu.async_copy(p_hbm.at[pl.ds(r0, _BATCH), :], pbufs[par],
                     sems.at[2 * par])
    pltpu.async_copy(r_hbm.at[pl.ds(r0, _BATCH), :], rbufs[par],
                     sems.at[2 * par + 1])

  def wait_batch(j, par):
    r0 = base + j * _BATCH
    pltpu.make_async_copy(p_hbm.at[pl.ds(r0, _BATCH), :], pbufs[par],
                          sems.at[2 * par]).wait()
    pltpu.make_async_copy(r_hbm.at[pl.ds(r0, _BATCH), :], rbufs[par],
                          sems.at[2 * par + 1]).wait()

  def process_batch(par, acc0):
    pb = pbufs[par]
    rb = rbufs[par]

    def row_step(r, acc):
      rsplat = izero + r

      m = minf
      lm = neg1
      for c in range(_NCHUNK):
        pv = plsc.load_gather(pb, [rsplat, cols[c]])
        rv = plsc.load_gather(rb, [rsplat, cols[c]])
        if c == _NCHUNK - 1:
          pv = jnp.where(tail_mask, pv, minf)
          rv = jnp.where(tail_mask, rv, neg1)
        m = jnp.maximum(m, pv)
        lm = jnp.maximum(lm, rv)
      pmax = _allreduce(m, jnp.maximum, lanes)
      thr = _allreduce(lm, jnp.minimum, lanes)

      s = jnp.zeros((_LANES,), jnp.float32)
      cnt = izero
      for c in range(_NCHUNK):
        e = jnp.exp(plsc.load_gather(pb, [rsplat, cols[c]]) - pmax)
        rv = plsc.load_gather(rb, [rsplat, cols[c]])
        if c == _NCHUNK - 1:
          e = jnp.where(tail_mask, e, 0.0)
          rv = jnp.where(tail_mask, rv, neg1)
        s = s + e
        sel = rv >= thr
        pos = posb + cnt
        plsc.store_scatter(cand_i, [pos], c * _LANES + lanes, mask=sel)
        cnt = cnt + sel.astype(jnp.int32)
      sumexp = _allreduce(s, jnp.add, lanes)

      iobuf[...] = _allreduce(cnt, jnp.maximum, lanes)
      maxcnt = iobuf[...][0]

      def cand_chunk(k):
        pos = posb + k
        ci = plsc.load_gather(cand_i, [pos])
        valid = k < cnt
        civ = jnp.where(valid, ci, 0)
        cv = plsc.load_gather(rb, [rsplat, civ])
        cv = jnp.where(valid, cv, neg1)
        return cv, ci

      def merge_into(run, cv, ci):
        run_v, run_i = run
        cvs, cis = plsc.sort_key_val(cv, ci, descending=False)
        take = cvs > run_v
        hv = jnp.where(take, cvs, run_v)
        hi = jnp.where(take, cis, run_i)
        nv, ni = plsc.sort_key_val(hv, hi, descending=True)
        return (nv, ni)

      runs = [(jnp.full((_LANES,), -2.0, jnp.float32), izero),
              (jnp.full((_LANES,), -2.0, jnp.float32), izero)]
      for k in range(_STATIC):
        cv, ci = cand_chunk(k)
        runs[k % 2] = merge_into(runs[k % 2], cv, ci)

      def mstep(k, carry):
        cv, ci = cand_chunk(k)
        return merge_into(carry, cv, ci)

      runs[0] = lax.fori_loop(_STATIC, maxcnt, mstep, runs[0])

      rev = 15 - lanes
      bvr = runs[1][0][rev]
      bir = runs[1][1][rev]
      take = bvr > runs[0][0]
      hv = jnp.where(take, bvr, runs[0][0])
      hi = jnp.where(take, bir, runs[0][1])
      top_v, top_i = plsc.sort_key_val(hv, hi, descending=True)

      rank = izero
      for j in range(_LANES):
        jv = izero + j
        bv = top_v[jv]
        bi = top_i[jv]
        beats = (bv > top_v) | ((bv == top_v) & (bi < top_i))
        rank = rank + beats.astype(jnp.int32)

      wr = wvec[rank]
      pv = plsc.load_gather(pb, [rsplat, top_i])
      soft = jnp.exp(pv - pmax) / sumexp
      dcg = _allreduce(top_v * soft * wr, jnp.add, lanes)
      idcg = _allreduce(top_v * wr, jnp.add, lanes)
      return acc + dcg / (idcg + 1e-8)

    return lax.fori_loop(0, _BATCH, row_step, acc0)

  copy_batch(0, 0)

  def pair_step(i, acc):
    wait_batch(2 * i, 0)
    copy_batch(2 * i + 1, 1)
    acc = process_batch(0, acc)
    wait_batch(2 * i + 1, 1)

    @pl.when(i + 1 < _NBATCH // 2)
    def _():
      copy_batch(2 * i + 2, 0)

    return process_batch(1, acc)

  acc = lax.fori_loop(0, _NBATCH // 2, pair_step,
                      jnp.zeros((_LANES,), jnp.float32))
  obuf[...] = acc
  pltpu.sync_copy(obuf, out_hbm.at[wid])


@jax.jit
def kernel(predictions, relevance_scores):
  positions = jnp.arange(_LANES, dtype=jnp.float32)
  wtable = jnp.where(positions < _K,
                     1.0 / jnp.log2(positions + 2.0), 0.0).astype(jnp.float32)

  mesh = plsc.VectorSubcoreMesh(core_axis_name="c", subcore_axis_name="s")
  partials = pl.kernel(
      _body,
      out_type=jax.ShapeDtypeStruct((_NWORKERS, _LANES), jnp.float32),
      mesh=mesh,
      compiler_params=pltpu.CompilerParams(needs_layout_passes=False),
      scratch_types=[
          pltpu.VMEM((_BATCH, _N), jnp.float32),
          pltpu.VMEM((_BATCH, _N), jnp.float32),
          pltpu.VMEM((_BATCH, _N), jnp.float32),
          pltpu.VMEM((_BATCH, _N), jnp.float32),
          pltpu.VMEM((_LANES * _REGION,), jnp.int32),
          pltpu.VMEM((_LANES,), jnp.float32),
          pltpu.VMEM((_LANES,), jnp.float32),
          pltpu.VMEM((_LANES,), jnp.int32),
          pltpu.SemaphoreType.DMA((4,)),
      ],
  )(
      predictions / _TEMPERATURE,
      relevance_scores,
      wtable,
  )
  return -jnp.sum(partials[:, 0]) / _ROWS


# speedup vs baseline: 1.0976x; 1.0976x over previous
import jax
import jax.numpy as jnp
from jax import lax
from jax.experimental import pallas as pl
from jax.experimental.pallas import tpu as pltpu
from jax.experimental.pallas import tpu_sc as plsc

_K = 10
_TEMPERATURE = 1.0
_ROWS = 16384
_N = 1000
_LANES = 16
_NCHUNK = 63
_NWORKERS = 32
_RPW = _ROWS // _NWORKERS
_BATCH = 16
_NBATCH = _RPW // _BATCH
_REGION = 64
_STATIC = 8


def _allreduce(v, op, lanes):
  for sh in (8, 4, 2, 1):
    v = op(v, v[jnp.bitwise_xor(lanes, sh)])
  return v


def _body(p_hbm, r_hbm, w_hbm, out_hbm,
          pbuf0, pbuf1, rbuf0, rbuf1, cand_i, wbuf, obuf, sems):
  pbufs = (pbuf0, pbuf1)
  rbufs = (rbuf0, rbuf1)
  wid = lax.axis_index("s") * 2 + lax.axis_index("c")
  base = wid * _RPW

  lanes = lax.iota(jnp.int32, _LANES)
  posb = lanes * _REGION
  tail_mask = lanes < 8
  minf = jnp.full((_LANES,), -jnp.inf, jnp.float32)
  neg1 = jnp.full((_LANES,), -1.0, jnp.float32)
  izero = jnp.zeros((_LANES,), jnp.int32)

  cols = [c * _LANES + lanes for c in range(_NCHUNK - 1)]
  cols.append(jnp.minimum((_NCHUNK - 1) * _LANES + lanes, _N - 1))

  pltpu.sync_copy(w_hbm, wbuf)
  wvec = wbuf[...]

  def copy_batch(j, par):
    r0 = base + j * _BATCH
    pltpu.async_copy(p_hbm.at[pl.ds(r0, _BATCH), :], pbufs[par],
                     sems.at[2 * par])
    pltpu.async_copy(r_hbm.at[pl.ds(r0, _BATCH), :], rbufs[par],
                     sems.at[2 * par + 1])

  def wait_batch(j, par):
    r0 = base + j * _BATCH
    pltpu.make_async_copy(p_hbm.at[pl.ds(r0, _BATCH), :], pbufs[par],
                          sems.at[2 * par]).wait()
    pltpu.make_async_copy(r_hbm.at[pl.ds(r0, _BATCH), :], rbufs[par],
                          sems.at[2 * par + 1]).wait()

  def process_batch(par, acc0):
    pb = pbufs[par]
    rb = rbufs[par]

    def row_step(r, acc):
      rsplat = izero + r

      m = minf
      lm = neg1
      for c in range(_NCHUNK):
        pv = plsc.load_gather(pb, [rsplat, cols[c]])
        rv = plsc.load_gather(rb, [rsplat, cols[c]])
        if c == _NCHUNK - 1:
          pv = jnp.where(tail_mask, pv, minf)
          rv = jnp.where(tail_mask, rv, neg1)
        m = jnp.maximum(m, pv)
        lm = jnp.maximum(lm, rv)
      pmax = _allreduce(m, jnp.maximum, lanes)
      thr = _allreduce(lm, jnp.minimum, lanes)

      s = jnp.zeros((_LANES,), jnp.float32)
      cnt = izero
      for c in range(_NCHUNK):
        e = jnp.exp(plsc.load_gather(pb, [rsplat, cols[c]]) - pmax)
        rv = plsc.load_gather(rb, [rsplat, cols[c]])
        if c == _NCHUNK - 1:
          e = jnp.where(tail_mask, e, 0.0)
          rv = jnp.where(tail_mask, rv, neg1)
        s = s + e
        sel = rv >= thr
        pos = posb + cnt
        plsc.store_scatter(cand_i, [pos], c * _LANES + lanes, mask=sel)
        cnt = cnt + sel.astype(jnp.int32)
      sumexp = _allreduce(s, jnp.add, lanes)

      def cand_chunk(k):
        pos = posb + k
        ci = plsc.load_gather(cand_i, [pos])
        valid = k < cnt
        civ = jnp.where(valid, ci, 0)
        cv = plsc.load_gather(rb, [rsplat, civ])
        cv = jnp.where(valid, cv, neg1)
        return cv, ci

      def merge_into(run, cv, ci):
        run_v, run_i = run
        cvs, cis = plsc.sort_key_val(cv, ci, descending=False)
        take = cvs > run_v
        hv = jnp.where(take, cvs, run_v)
        hi = jnp.where(take, cis, run_i)
        nv, ni = plsc.sort_key_val(hv, hi, descending=True)
        return (nv, ni)

      runs = [(jnp.full((_LANES,), -2.0, jnp.float32), izero),
              (jnp.full((_LANES,), -2.0, jnp.float32), izero)]
      for k in range(_STATIC):
        cv, ci = cand_chunk(k)
        runs[k % 2] = merge_into(runs[k % 2], cv, ci)

      def wcond(carry):
        k, _, _ = carry
        return jnp.any(cnt > k)

      def wbody(carry):
        k, run_v, run_i = carry
        cv, ci = cand_chunk(k)
        nv, ni = merge_into((run_v, run_i), cv, ci)
        return (k + 1, nv, ni)

      _, rv0, ri0 = lax.while_loop(
          wcond, wbody, (jnp.int32(_STATIC), runs[0][0], runs[0][1]))
      runs[0] = (rv0, ri0)

      rev = 15 - lanes
      bvr = runs[1][0][rev]
      bir = runs[1][1][rev]
      take = bvr > runs[0][0]
      hv = jnp.where(take, bvr, runs[0][0])
      hi = jnp.where(take, bir, runs[0][1])
      top_v, top_i = plsc.sort_key_val(hv, hi, descending=True)

      rank = izero
      for j in range(_LANES):
        jv = izero + j
        bv = top_v[jv]
        bi = top_i[jv]
        beats = (bv > top_v) | ((bv == top_v) & (bi < top_i))
        rank = rank + beats.astype(jnp.int32)

      wr = wvec[rank]
      pv = plsc.load_gather(pb, [rsplat, top_i])
      soft = jnp.exp(pv - pmax) / sumexp
      dcg = _allreduce(top_v * soft * wr, jnp.add, lanes)
      idcg = _allreduce(top_v * wr, jnp.add, lanes)
      return acc + dcg / (idcg + 1e-8)

    return lax.fori_loop(0, _BATCH, row_step, acc0)

  copy_batch(0, 0)

  def pair_step(i, acc):
    wait_batch(2 * i, 0)
    copy_batch(2 * i + 1, 1)
    acc = process_batch(0, acc)
    wait_batch(2 * i + 1, 1)

    @pl.when(i + 1 < _NBATCH // 2)
    def _():
      copy_batch(2 * i + 2, 0)

    return process_batch(1, acc)

  acc = lax.fori_loop(0, _NBATCH // 2, pair_step,
                      jnp.zeros((_LANES,), jnp.float32))
  obuf[...] = acc
  pltpu.sync_copy(obuf, out_hbm.at[wid])


@jax.jit
def kernel(predictions, relevance_scores):
  positions = jnp.arange(_LANES, dtype=jnp.float32)
  wtable = jnp.where(positions < _K,
                     1.0 / jnp.log2(positions + 2.0), 0.0).astype(jnp.float32)

  mesh = plsc.VectorSubcoreMesh(core_axis_name="c", subcore_axis_name="s")
  partials = pl.kernel(
      _body,
      out_type=jax.ShapeDtypeStruct((_NWORKERS, _LANES), jnp.float32),
      mesh=mesh,
      compiler_params=pltpu.CompilerParams(needs_layout_passes=False),
      scratch_types=[
          pltpu.VMEM((_BATCH, _N), jnp.float32),
          pltpu.VMEM((_BATCH, _N), jnp.float32),
          pltpu.VMEM((_BATCH, _N), jnp.float32),
          pltpu.VMEM((_BATCH, _N), jnp.float32),
          pltpu.VMEM((_LANES * _REGION,), jnp.int32),
          pltpu.VMEM((_LANES,), jnp.float32),
          pltpu.VMEM((_LANES,), jnp.float32),
          pltpu.SemaphoreType.DMA((4,)),
      ],
  )(
      predictions / _TEMPERATURE,
      relevance_scores,
      wtable,
  )
  return -jnp.sum(partials[:, 0]) / _ROWS


# speedup vs baseline: 1.4008x; 1.2762x over previous
import jax
import jax.numpy as jnp
from jax import lax
from jax.experimental import pallas as pl
from jax.experimental.pallas import tpu as pltpu
from jax.experimental.pallas import tpu_sc as plsc

_K = 10
_TEMPERATURE = 1.0
_ROWS = 16384
_N = 1000
_LANES = 16
_NCHUNK = 63
_NWORKERS = 32
_RPW = _ROWS // _NWORKERS
_BATCH = 16
_NBATCH = _RPW // _BATCH
_NSTREAM = 8


def _allreduce(v, op, lanes):
  for sh in (8, 4, 2, 1):
    v = op(v, v[jnp.bitwise_xor(lanes, sh)])
  return v


def _merge_desc(av, ai, bv, bi, rev):
  bvr = bv[rev]
  bir = bi[rev]
  take = bvr > av
  hv = jnp.where(take, bvr, av)
  hi = jnp.where(take, bir, ai)
  nv, ni = plsc.sort_key_val(hv, hi, descending=True)
  return nv, ni


def _body(p_hbm, r_hbm, w_hbm, out_hbm,
          pbuf0, pbuf1, rbuf0, rbuf1, wbuf, obuf, sems):
  pbufs = (pbuf0, pbuf1)
  rbufs = (rbuf0, rbuf1)
  wid = lax.axis_index("s") * 2 + lax.axis_index("c")
  base = wid * _RPW

  lanes = lax.iota(jnp.int32, _LANES)
  rev = 15 - lanes
  tail_mask = lanes < 8
  minf = jnp.full((_LANES,), -jnp.inf, jnp.float32)
  neg1 = jnp.full((_LANES,), -1.0, jnp.float32)

  cols = [c * _LANES + lanes for c in range(_NCHUNK - 1)]
  cols.append(jnp.minimum((_NCHUNK - 1) * _LANES + lanes, _N - 1))

  pltpu.sync_copy(w_hbm, wbuf)
  wvec = wbuf[...]

  def copy_batch(j, par):
    r0 = base + j * _BATCH
    pltpu.async_copy(p_hbm.at[pl.ds(r0, _BATCH), :], pbufs[par],
                     sems.at[2 * par])
    pltpu.async_copy(r_hbm.at[pl.ds(r0, _BATCH), :], rbufs[par],
                     sems.at[2 * par + 1])

  def wait_batch(j, par):
    r0 = base + j * _BATCH
    pltpu.make_async_copy(p_hbm.at[pl.ds(r0, _BATCH), :], pbufs[par],
                          sems.at[2 * par]).wait()
    pltpu.make_async_copy(r_hbm.at[pl.ds(r0, _BATCH), :], rbufs[par],
                          sems.at[2 * par + 1]).wait()

  def process_batch(par, acc0):
    pb = pbufs[par]
    rb = rbufs[par]

    def row_step(r, acc):
      rsplat = jnp.full((_LANES,), 0, jnp.int32) + r

      m = minf
      s = jnp.zeros((_LANES,), jnp.float32)
      run_v = [jnp.full((_LANES,), -2.0, jnp.float32)] * _NSTREAM
      run_i = [jnp.zeros((_LANES,), jnp.int32)] * _NSTREAM
      for c in range(_NCHUNK):
        st = c % _NSTREAM
        cv = plsc.load_gather(rb, [rsplat, cols[c]])
        pv = plsc.load_gather(pb, [rsplat, cols[c]])
        if c == _NCHUNK - 1:
          cv = jnp.where(tail_mask, cv, neg1)
          pv = jnp.where(tail_mask, pv, minf)
        ci = c * _LANES + lanes
        cvs, cis = plsc.sort_key_val(cv, ci, descending=False)
        take = cvs > run_v[st]
        hv = jnp.where(take, cvs, run_v[st])
        hi = jnp.where(take, cis, run_i[st])
        nv, ni = plsc.sort_key_val(hv, hi, descending=True)
        run_v[st], run_i[st] = nv, ni
        m2 = jnp.maximum(m, pv)
        s = s * jnp.exp(m - m2) + jnp.exp(pv - m2)
        m = m2

      pmax = _allreduce(m, jnp.maximum, lanes)
      sumexp = _allreduce(s * jnp.exp(m - pmax), jnp.add, lanes)

      while len(run_v) > 1:
        nxt_v, nxt_i = [], []
        for a in range(0, len(run_v), 2):
          mv, mi = _merge_desc(run_v[a], run_i[a], run_v[a + 1], run_i[a + 1],
                               rev)
          nxt_v.append(mv)
          nxt_i.append(mi)
        run_v, run_i = nxt_v, nxt_i
      top_v, top_i = run_v[0], run_i[0]

      rank = jnp.zeros((_LANES,), jnp.int32)
      for j in range(_LANES):
        jv = jnp.full((_LANES,), j, jnp.int32)
        bv = top_v[jv]
        bi = top_i[jv]
        beats = (bv > top_v) | ((bv == top_v) & (bi < top_i))
        rank = rank + beats.astype(jnp.int32)

      wr = wvec[rank]
      pv = plsc.load_gather(pb, [rsplat, top_i])
      soft = jnp.exp(pv - pmax) / sumexp
      dcg = _allreduce(top_v * soft * wr, jnp.add, lanes)
      idcg = _allreduce(top_v * wr, jnp.add, lanes)
      return acc + dcg / (idcg + 1e-8)

    return lax.fori_loop(0, _BATCH, row_step, acc0)

  copy_batch(0, 0)

  def pair_step(i, acc):
    wait_batch(2 * i, 0)
    copy_batch(2 * i + 1, 1)
    acc = process_batch(0, acc)
    wait_batch(2 * i + 1, 1)

    @pl.when(i + 1 < _NBATCH // 2)
    def _():
      copy_batch(2 * i + 2, 0)

    return process_batch(1, acc)

  acc = lax.fori_loop(0, _NBATCH // 2, pair_step,
                      jnp.zeros((_LANES,), jnp.float32))
  obuf[...] = acc
  pltpu.sync_copy(obuf, out_hbm.at[wid])


@jax.jit
def kernel(predictions, relevance_scores):
  positions = jnp.arange(_LANES, dtype=jnp.float32)
  wtable = jnp.where(positions < _K,
                     1.0 / jnp.log2(positions + 2.0), 0.0).astype(jnp.float32)

  mesh = plsc.VectorSubcoreMesh(core_axis_name="c", subcore_axis_name="s")
  partials = pl.kernel(
      _body,
      out_type=jax.ShapeDtypeStruct((_NWORKERS, _LANES), jnp.float32),
      mesh=mesh,
      compiler_params=pltpu.CompilerParams(needs_layout_passes=False),
      scratch_types=[
          pltpu.VMEM((_BATCH, _N), jnp.float32),
          pltpu.VMEM((_BATCH, _N), jnp.float32),
          pltpu.VMEM((_BATCH, _N), jnp.float32),
          pltpu.VMEM((_BATCH, _N), jnp.float32),
          pltpu.VMEM((_LANES,), jnp.float32),
          pltpu.VMEM((_LANES,), jnp.float32),
          pltpu.SemaphoreType.DMA((4,)),
      ],
  )(
      predictions / _TEMPERATURE,
      relevance_scores,
      wtable,
  )
  return -jnp.sum(partials[:, 0]) / _ROWS


# speedup vs baseline: 1.4410x; 1.0287x over previous
import jax
import jax.numpy as jnp
from jax import lax
from jax.experimental import pallas as pl
from jax.experimental.pallas import tpu as pltpu
from jax.experimental.pallas import tpu_sc as plsc

_K = 10
_TEMPERATURE = 1.0
_ROWS = 16384
_N = 1000
_LANES = 16
_NCHUNK = 63
_NWORKERS = 32
_RPW = _ROWS // _NWORKERS
_BATCH = 16
_NBATCH = _RPW // _BATCH
_NSTREAM = 4


def _allreduce(v, op, lanes):
  for sh in (8, 4, 2, 1):
    v = op(v, v[jnp.bitwise_xor(lanes, sh)])
  return v


def _merge_desc(av, ai, bv, bi, rev):
  bvr = bv[rev]
  bir = bi[rev]
  take = bvr > av
  hv = jnp.where(take, bvr, av)
  hi = jnp.where(take, bir, ai)
  nv, ni = plsc.sort_key_val(hv, hi, descending=True)
  return nv, ni


def _body(p_hbm, r_hbm, w_hbm, out_hbm,
          pbuf0, pbuf1, rbuf0, rbuf1, wbuf, obuf, sems):
  pbufs = (pbuf0, pbuf1)
  rbufs = (rbuf0, rbuf1)
  wid = lax.axis_index("s") * 2 + lax.axis_index("c")
  base = wid * _RPW

  lanes = lax.iota(jnp.int32, _LANES)
  rev = 15 - lanes
  tail_mask = lanes < 8
  minf = jnp.full((_LANES,), -jnp.inf, jnp.float32)
  neg1 = jnp.full((_LANES,), -1.0, jnp.float32)

  cols = [c * _LANES + lanes for c in range(_NCHUNK - 1)]
  cols.append(jnp.minimum((_NCHUNK - 1) * _LANES + lanes, _N - 1))

  pltpu.sync_copy(w_hbm, wbuf)
  wvec = wbuf[...]

  def copy_batch(j, par):
    r0 = base + j * _BATCH
    pltpu.async_copy(p_hbm.at[pl.ds(r0, _BATCH), :], pbufs[par],
                     sems.at[2 * par])
    pltpu.async_copy(r_hbm.at[pl.ds(r0, _BATCH), :], rbufs[par],
                     sems.at[2 * par + 1])

  def wait_batch(j, par):
    r0 = base + j * _BATCH
    pltpu.make_async_copy(p_hbm.at[pl.ds(r0, _BATCH), :], pbufs[par],
                          sems.at[2 * par]).wait()
    pltpu.make_async_copy(r_hbm.at[pl.ds(r0, _BATCH), :], rbufs[par],
                          sems.at[2 * par + 1]).wait()

  def process_batch(par, acc0):
    pb = pbufs[par]
    rb = rbufs[par]

    def row_step(r, acc):
      rsplat = jnp.full((_LANES,), 0, jnp.int32) + r

      m = minf
      s = jnp.zeros((_LANES,), jnp.float32)
      run_v = [jnp.full((_LANES,), -2.0, jnp.float32)] * _NSTREAM
      run_i = [jnp.zeros((_LANES,), jnp.int32)] * _NSTREAM
      for c in range(_NCHUNK):
        st = c % _NSTREAM
        cv = plsc.load_gather(rb, [rsplat, cols[c]])
        pv = plsc.load_gather(pb, [rsplat, cols[c]])
        if c == _NCHUNK - 1:
          cv = jnp.where(tail_mask, cv, neg1)
          pv = jnp.where(tail_mask, pv, minf)
        ci = c * _LANES + lanes
        cvs, cis = plsc.sort_key_val(cv, ci, descending=False)
        take = cvs > run_v[st]
        hv = jnp.where(take, cvs, run_v[st])
        hi = jnp.where(take, cis, run_i[st])
        nv, ni = plsc.sort_key_val(hv, hi, descending=True)
        run_v[st], run_i[st] = nv, ni
        m2 = jnp.maximum(m, pv)
        s = s * jnp.exp(m - m2) + jnp.exp(pv - m2)
        m = m2

      pmax = _allreduce(m, jnp.maximum, lanes)
      sumexp = _allreduce(s * jnp.exp(m - pmax), jnp.add, lanes)

      while len(run_v) > 1:
        nxt_v, nxt_i = [], []
        for a in range(0, len(run_v), 2):
          mv, mi = _merge_desc(run_v[a], run_i[a], run_v[a + 1], run_i[a + 1],
                               rev)
          nxt_v.append(mv)
          nxt_i.append(mi)
        run_v, run_i = nxt_v, nxt_i
      top_v, top_i = run_v[0], run_i[0]

      rank = jnp.zeros((_LANES,), jnp.int32)
      for j in range(_LANES):
        jv = jnp.full((_LANES,), j, jnp.int32)
        bv = top_v[jv]
        bi = top_i[jv]
        beats = (bv > top_v) | ((bv == top_v) & (bi < top_i))
        rank = rank + beats.astype(jnp.int32)

      wr = wvec[rank]
      pv = plsc.load_gather(pb, [rsplat, top_i])
      soft = jnp.exp(pv - pmax) / sumexp
      dcg = _allreduce(top_v * soft * wr, jnp.add, lanes)
      idcg = _allreduce(top_v * wr, jnp.add, lanes)
      return acc + dcg / (idcg + 1e-8)

    return lax.fori_loop(0, _BATCH, row_step, acc0)

  copy_batch(0, 0)

  def pair_step(i, acc):
    wait_batch(2 * i, 0)
    copy_batch(2 * i + 1, 1)
    acc = process_batch(0, acc)
    wait_batch(2 * i + 1, 1)

    @pl.when(i + 1 < _NBATCH // 2)
    def _():
      copy_batch(2 * i + 2, 0)

    return process_batch(1, acc)

  acc = lax.fori_loop(0, _NBATCH // 2, pair_step,
                      jnp.zeros((_LANES,), jnp.float32))
  obuf[...] = acc
  pltpu.sync_copy(obuf, out_hbm.at[wid])


@jax.jit
def kernel(predictions, relevance_scores):
  positions = jnp.arange(_LANES, dtype=jnp.float32)
  wtable = jnp.where(positions < _K,
                     1.0 / jnp.log2(positions + 2.0), 0.0).astype(jnp.float32)

  mesh = plsc.VectorSubcoreMesh(core_axis_name="c", subcore_axis_name="s")
  partials = pl.kernel(
      _body,
      out_type=jax.ShapeDtypeStruct((_NWORKERS, _LANES), jnp.float32),
      mesh=mesh,
      compiler_params=pltpu.CompilerParams(needs_layout_passes=False),
      scratch_types=[
          pltpu.VMEM((_BATCH, _N), jnp.float32),
          pltpu.VMEM((_BATCH, _N), jnp.float32),
          pltpu.VMEM((_BATCH, _N), jnp.float32),
          pltpu.VMEM((_BATCH, _N), jnp.float32),
          pltpu.VMEM((_LANES,), jnp.float32),
          pltpu.VMEM((_LANES,), jnp.float32),
          pltpu.SemaphoreType.DMA((4,)),
      ],
  )(
      predictions / _TEMPERATURE,
      relevance_scores,
      wtable,
  )
  return -jnp.sum(partials[:, 0]) / _ROWS


# speedup vs baseline: 1.4626x; 1.0150x over previous
import jax
import jax.numpy as jnp
from jax import lax
from jax.experimental import pallas as pl
from jax.experimental.pallas import tpu as pltpu
from jax.experimental.pallas import tpu_sc as plsc

_K = 10
_TEMPERATURE = 1.0
_ROWS = 16384
_N = 1000
_LANES = 16
_NCHUNK = 63
_NWORKERS = 32
_RPW = _ROWS // _NWORKERS
_BATCH = 16
_NBATCH = _RPW // _BATCH
_NSTREAM = 2


def _allreduce(v, op, lanes):
  for sh in (8, 4, 2, 1):
    v = op(v, v[jnp.bitwise_xor(lanes, sh)])
  return v


def _merge_desc(av, ai, bv, bi, rev):
  bvr = bv[rev]
  bir = bi[rev]
  take = bvr > av
  hv = jnp.where(take, bvr, av)
  hi = jnp.where(take, bir, ai)
  nv, ni = plsc.sort_key_val(hv, hi, descending=True)
  return nv, ni


def _body(p_hbm, r_hbm, w_hbm, out_hbm,
          pbuf0, pbuf1, rbuf0, rbuf1, wbuf, obuf, sems):
  pbufs = (pbuf0, pbuf1)
  rbufs = (rbuf0, rbuf1)
  wid = lax.axis_index("s") * 2 + lax.axis_index("c")
  base = wid * _RPW

  lanes = lax.iota(jnp.int32, _LANES)
  rev = 15 - lanes
  tail_mask = lanes < 8
  minf = jnp.full((_LANES,), -jnp.inf, jnp.float32)
  neg1 = jnp.full((_LANES,), -1.0, jnp.float32)

  cols = [c * _LANES + lanes for c in range(_NCHUNK - 1)]
  cols.append(jnp.minimum((_NCHUNK - 1) * _LANES + lanes, _N - 1))

  pltpu.sync_copy(w_hbm, wbuf)
  wvec = wbuf[...]

  def copy_batch(j, par):
    r0 = base + j * _BATCH
    pltpu.async_copy(p_hbm.at[pl.ds(r0, _BATCH), :], pbufs[par],
                     sems.at[2 * par])
    pltpu.async_copy(r_hbm.at[pl.ds(r0, _BATCH), :], rbufs[par],
                     sems.at[2 * par + 1])

  def wait_batch(j, par):
    r0 = base + j * _BATCH
    pltpu.make_async_copy(p_hbm.at[pl.ds(r0, _BATCH), :], pbufs[par],
                          sems.at[2 * par]).wait()
    pltpu.make_async_copy(r_hbm.at[pl.ds(r0, _BATCH), :], rbufs[par],
                          sems.at[2 * par + 1]).wait()

  def process_batch(par, acc0):
    pb = pbufs[par]
    rb = rbufs[par]

    def row_step(r, acc):
      rsplat = jnp.full((_LANES,), 0, jnp.int32) + r

      m = minf
      s = jnp.zeros((_LANES,), jnp.float32)
      run_v = [jnp.full((_LANES,), -2.0, jnp.float32)] * _NSTREAM
      run_i = [jnp.zeros((_LANES,), jnp.int32)] * _NSTREAM
      for c in range(_NCHUNK):
        st = c % _NSTREAM
        cv = plsc.load_gather(rb, [rsplat, cols[c]])
        pv = plsc.load_gather(pb, [rsplat, cols[c]])
        if c == _NCHUNK - 1:
          cv = jnp.where(tail_mask, cv, neg1)
          pv = jnp.where(tail_mask, pv, minf)
        ci = c * _LANES + lanes
        cvs, cis = plsc.sort_key_val(cv, ci, descending=False)
        take = cvs > run_v[st]
        hv = jnp.where(take, cvs, run_v[st])
        hi = jnp.where(take, cis, run_i[st])
        nv, ni = plsc.sort_key_val(hv, hi, descending=True)
        run_v[st], run_i[st] = nv, ni
        m2 = jnp.maximum(m, pv)
        s = s * jnp.exp(m - m2) + jnp.exp(pv - m2)
        m = m2

      pmax = _allreduce(m, jnp.maximum, lanes)
      sumexp = _allreduce(s * jnp.exp(m - pmax), jnp.add, lanes)

      while len(run_v) > 1:
        nxt_v, nxt_i = [], []
        for a in range(0, len(run_v), 2):
          mv, mi = _merge_desc(run_v[a], run_i[a], run_v[a + 1], run_i[a + 1],
                               rev)
          nxt_v.append(mv)
          nxt_i.append(mi)
        run_v, run_i = nxt_v, nxt_i
      top_v, top_i = run_v[0], run_i[0]

      rank = jnp.zeros((_LANES,), jnp.int32)
      for j in range(_LANES):
        jv = jnp.full((_LANES,), j, jnp.int32)
        bv = top_v[jv]
        bi = top_i[jv]
        beats = (bv > top_v) | ((bv == top_v) & (bi < top_i))
        rank = rank + beats.astype(jnp.int32)

      wr = wvec[rank]
      pv = plsc.load_gather(pb, [rsplat, top_i])
      soft = jnp.exp(pv - pmax) / sumexp
      dcg = _allreduce(top_v * soft * wr, jnp.add, lanes)
      idcg = _allreduce(top_v * wr, jnp.add, lanes)
      return acc + dcg / (idcg + 1e-8)

    return lax.fori_loop(0, _BATCH, row_step, acc0)

  copy_batch(0, 0)

  def pair_step(i, acc):
    wait_batch(2 * i, 0)
    copy_batch(2 * i + 1, 1)
    acc = process_batch(0, acc)
    wait_batch(2 * i + 1, 1)

    @pl.when(i + 1 < _NBATCH // 2)
    def _():
      copy_batch(2 * i + 2, 0)

    return process_batch(1, acc)

  acc = lax.fori_loop(0, _NBATCH // 2, pair_step,
                      jnp.zeros((_LANES,), jnp.float32))
  obuf[...] = acc
  pltpu.sync_copy(obuf, out_hbm.at[wid])


@jax.jit
def kernel(predictions, relevance_scores):
  positions = jnp.arange(_LANES, dtype=jnp.float32)
  wtable = jnp.where(positions < _K,
                     1.0 / jnp.log2(positions + 2.0), 0.0).astype(jnp.float32)

  mesh = plsc.VectorSubcoreMesh(core_axis_name="c", subcore_axis_name="s")
  partials = pl.kernel(
      _body,
      out_type=jax.ShapeDtypeStruct((_NWORKERS, _LANES), jnp.float32),
      mesh=mesh,
      compiler_params=pltpu.CompilerParams(needs_layout_passes=False),
      scratch_types=[
          pltpu.VMEM((_BATCH, _N), jnp.float32),
          pltpu.VMEM((_BATCH, _N), jnp.float32),
          pltpu.VMEM((_BATCH, _N), jnp.float32),
          pltpu.VMEM((_BATCH, _N), jnp.float32),
          pltpu.VMEM((_LANES,), jnp.float32),
          pltpu.VMEM((_LANES,), jnp.float32),
          pltpu.SemaphoreType.DMA((4,)),
      ],
  )(
      predictions / _TEMPERATURE,
      relevance_scores,
      wtable,
  )
  return -jnp.sum(partials[:, 0]) / _ROWS


# speedup vs baseline: 1.4893x; 1.0183x over previous
import jax
import jax.numpy as jnp
from jax import lax
from jax.experimental import pallas as pl
from jax.experimental.pallas import tpu as pltpu
from jax.experimental.pallas import tpu_sc as plsc

_K = 10
_TEMPERATURE = 1.0
_ROWS = 16384
_N = 1000
_LANES = 16
_NCHUNK = 63
_NWORKERS = 32
_RPW = _ROWS // _NWORKERS
_BATCH = 16
_NBATCH = _RPW // _BATCH
_NSTREAM = 1


def _allreduce(v, op, lanes):
  for sh in (8, 4, 2, 1):
    v = op(v, v[jnp.bitwise_xor(lanes, sh)])
  return v


def _merge_desc(av, ai, bv, bi, rev):
  bvr = bv[rev]
  bir = bi[rev]
  take = bvr > av
  hv = jnp.where(take, bvr, av)
  hi = jnp.where(take, bir, ai)
  nv, ni = plsc.sort_key_val(hv, hi, descending=True)
  return nv, ni


def _body(p_hbm, r_hbm, w_hbm, out_hbm,
          pbuf0, pbuf1, rbuf0, rbuf1, wbuf, obuf, sems):
  pbufs = (pbuf0, pbuf1)
  rbufs = (rbuf0, rbuf1)
  wid = lax.axis_index("s") * 2 + lax.axis_index("c")
  base = wid * _RPW

  lanes = lax.iota(jnp.int32, _LANES)
  rev = 15 - lanes
  tail_mask = lanes < 8
  minf = jnp.full((_LANES,), -jnp.inf, jnp.float32)
  neg1 = jnp.full((_LANES,), -1.0, jnp.float32)

  cols = [c * _LANES + lanes for c in range(_NCHUNK - 1)]
  cols.append(jnp.minimum((_NCHUNK - 1) * _LANES + lanes, _N - 1))

  pltpu.sync_copy(w_hbm, wbuf)
  wvec = wbuf[...]

  def copy_batch(j, par):
    r0 = base + j * _BATCH
    pltpu.async_copy(p_hbm.at[pl.ds(r0, _BATCH), :], pbufs[par],
                     sems.at[2 * par])
    pltpu.async_copy(r_hbm.at[pl.ds(r0, _BATCH), :], rbufs[par],
                     sems.at[2 * par + 1])

  def wait_batch(j, par):
    r0 = base + j * _BATCH
    pltpu.make_async_copy(p_hbm.at[pl.ds(r0, _BATCH), :], pbufs[par],
                          sems.at[2 * par]).wait()
    pltpu.make_async_copy(r_hbm.at[pl.ds(r0, _BATCH), :], rbufs[par],
                          sems.at[2 * par + 1]).wait()

  def process_batch(par, acc0):
    pb = pbufs[par]
    rb = rbufs[par]

    def row_step(r, acc):
      rsplat = jnp.full((_LANES,), 0, jnp.int32) + r

      m = minf
      s = jnp.zeros((_LANES,), jnp.float32)
      run_v = [jnp.full((_LANES,), -2.0, jnp.float32)] * _NSTREAM
      run_i = [jnp.zeros((_LANES,), jnp.int32)] * _NSTREAM
      for c in range(_NCHUNK):
        st = c % _NSTREAM
        cv = plsc.load_gather(rb, [rsplat, cols[c]])
        pv = plsc.load_gather(pb, [rsplat, cols[c]])
        if c == _NCHUNK - 1:
          cv = jnp.where(tail_mask, cv, neg1)
          pv = jnp.where(tail_mask, pv, minf)
        ci = c * _LANES + lanes
        cvs, cis = plsc.sort_key_val(cv, ci, descending=False)
        take = cvs > run_v[st]
        hv = jnp.where(take, cvs, run_v[st])
        hi = jnp.where(take, cis, run_i[st])
        nv, ni = plsc.sort_key_val(hv, hi, descending=True)
        run_v[st], run_i[st] = nv, ni
        m2 = jnp.maximum(m, pv)
        s = s * jnp.exp(m - m2) + jnp.exp(pv - m2)
        m = m2

      pmax = _allreduce(m, jnp.maximum, lanes)
      sumexp = _allreduce(s * jnp.exp(m - pmax), jnp.add, lanes)

      while len(run_v) > 1:
        nxt_v, nxt_i = [], []
        for a in range(0, len(run_v), 2):
          mv, mi = _merge_desc(run_v[a], run_i[a], run_v[a + 1], run_i[a + 1],
                               rev)
          nxt_v.append(mv)
          nxt_i.append(mi)
        run_v, run_i = nxt_v, nxt_i
      top_v, top_i = run_v[0], run_i[0]

      rank = jnp.zeros((_LANES,), jnp.int32)
      for j in range(_LANES):
        jv = jnp.full((_LANES,), j, jnp.int32)
        bv = top_v[jv]
        bi = top_i[jv]
        beats = (bv > top_v) | ((bv == top_v) & (bi < top_i))
        rank = rank + beats.astype(jnp.int32)

      wr = wvec[rank]
      pv = plsc.load_gather(pb, [rsplat, top_i])
      soft = jnp.exp(pv - pmax) / sumexp
      dcg = _allreduce(top_v * soft * wr, jnp.add, lanes)
      idcg = _allreduce(top_v * wr, jnp.add, lanes)
      return acc + dcg / (idcg + 1e-8)

    return lax.fori_loop(0, _BATCH, row_step, acc0)

  copy_batch(0, 0)

  def pair_step(i, acc):
    wait_batch(2 * i, 0)
    copy_batch(2 * i + 1, 1)
    acc = process_batch(0, acc)
    wait_batch(2 * i + 1, 1)

    @pl.when(i + 1 < _NBATCH // 2)
    def _():
      copy_batch(2 * i + 2, 0)

    return process_batch(1, acc)

  acc = lax.fori_loop(0, _NBATCH // 2, pair_step,
                      jnp.zeros((_LANES,), jnp.float32))
  obuf[...] = acc
  pltpu.sync_copy(obuf, out_hbm.at[wid])


@jax.jit
def kernel(predictions, relevance_scores):
  positions = jnp.arange(_LANES, dtype=jnp.float32)
  wtable = jnp.where(positions < _K,
                     1.0 / jnp.log2(positions + 2.0), 0.0).astype(jnp.float32)

  mesh = plsc.VectorSubcoreMesh(core_axis_name="c", subcore_axis_name="s")
  partials = pl.kernel(
      _body,
      out_type=jax.ShapeDtypeStruct((_NWORKERS, _LANES), jnp.float32),
      mesh=mesh,
      compiler_params=pltpu.CompilerParams(needs_layout_passes=False),
      scratch_types=[
          pltpu.VMEM((_BATCH, _N), jnp.float32),
          pltpu.VMEM((_BATCH, _N), jnp.float32),
          pltpu.VMEM((_BATCH, _N), jnp.float32),
          pltpu.VMEM((_BATCH, _N), jnp.float32),
          pltpu.VMEM((_LANES,), jnp.float32),
          pltpu.VMEM((_LANES,), jnp.float32),
          pltpu.SemaphoreType.DMA((4,)),
      ],
  )(
      predictions / _TEMPERATURE,
      relevance_scores,
      wtable,
  )
  return -jnp.sum(partials[:, 0]) / _ROWS
